# baseline (speedup 1.0000x reference)
import functools

import jax
import jax.numpy as jnp
from jax import lax
from jax.experimental import pallas as pl
from jax.experimental.pallas import tpu as pltpu
from jax.experimental.pallas import tpu_sc as plsc

N = 10000
E = 320000
HALF = E // 2
H = 128
R = 200
NC, NS = 2, 16
EPT = HALF // NS
K = 80
NCHUNK = EPT // K
WT = 10
WR = N // WT
DW = 128

_mesh = plsc.VectorSubcoreMesh(
    core_axis_name="c", subcore_axis_name="s", num_cores=NC, num_subcores=NS)


@functools.partial(
    pl.kernel,
    out_type=jax.ShapeDtypeStruct((NC, N, DW), jnp.float32),
    mesh=_mesh,
    scratch_types=[
        pltpu.VMEM_SHARED((N, DW), jnp.float32),
        pltpu.VMEM((EPT,), jnp.int32),
        pltpu.VMEM((K,), jnp.int32),
        pltpu.VMEM((K,), jnp.int32),
        pltpu.VMEM((K, DW), jnp.float32),
        pltpu.SemaphoreType.DMA,
        pltpu.SemaphoreType.DMA,
    ],
)
def _deg_kernel(row_hbm, ones_hbm, zeros_hbm, out_hbm,
                deg_sh, row_flat, row_v0, row_v1, ones_v, semA, semB):
    c = lax.axis_index("c")
    s = lax.axis_index("s")

    @pl.when(s < WT)
    def _():
        pltpu.sync_copy(zeros_hbm, deg_sh.at[pl.ds(s * WR, WR)])

    pltpu.sync_copy(ones_hbm, ones_v)
    pltpu.sync_copy(row_hbm.at[pl.ds(c * HALF + s * EPT, EPT)], row_flat)
    plsc.subcore_barrier()

    def fill(dst, i):
        for j in range(K // 16):
            dst[pl.ds(j * 16, 16)] = row_flat[pl.ds(i * K + j * 16, 16)]

    def start(buf, sem):
        pltpu.async_copy(ones_v, deg_sh.at[buf], sem, add=True)

    def drain(buf, sem):
        pltpu.make_async_copy(ones_v, deg_sh.at[buf], sem).wait()

    fill(row_v0, 0)
    start(row_v0, semA)

    def chunk(i, carry):
        @pl.when(i % 2 == 0)
        def _():
            @pl.when(i + 1 < NCHUNK)
            def _():
                fill(row_v1, i + 1)
                start(row_v1, semB)
            drain(row_v0, semA)

        @pl.when(i % 2 == 1)
        def _():
            @pl.when(i + 1 < NCHUNK)
            def _():
                fill(row_v0, i + 1)
                start(row_v0, semA)
            drain(row_v1, semB)

        return carry

    lax.fori_loop(0, NCHUNK, chunk, 0)
    plsc.subcore_barrier()

    @pl.when(s < WT)
    def _():
        pltpu.sync_copy(deg_sh.at[pl.ds(s * WR, WR)],
                        out_hbm.at[c, pl.ds(s * WR, WR)])


@functools.partial(
    pl.kernel,
    out_type=jax.ShapeDtypeStruct((NC, N, H), jnp.float32),
    mesh=_mesh,
    scratch_types=[
        pltpu.VMEM_SHARED((N, H), jnp.float32),
        pltpu.VMEM((K,), jnp.int32),
        pltpu.VMEM((K,), jnp.int32),
        pltpu.VMEM((K,), jnp.int32),
        pltpu.VMEM((K,), jnp.int32),
        pltpu.VMEM((K,), jnp.int32),
        pltpu.VMEM((K,), jnp.int32),
        pltpu.VMEM((R, H), jnp.float32),
        pltpu.VMEM((K, H), jnp.float32),
        pltpu.VMEM((K, H), jnp.float32),
        pltpu.SemaphoreType.DMA,
        pltpu.SemaphoreType.DMA,
        pltpu.SemaphoreType.DMA,
        pltpu.SemaphoreType.DMA,
    ],
)
def _agg_kernel(row_hbm, col2_hbm, etype_hbm, xcat_hbm, rel_hbm, zeros_hbm,
                out_hbm, agg_sh, col0, row0, typ0, col1, row1, typ1,
                rel_v, xr0, xr1, semg0, semg1, semi0, semi1):
    c = lax.axis_index("c")
    s = lax.axis_index("s")

    @pl.when(s < WT)
    def _():
        pltpu.sync_copy(zeros_hbm, agg_sh.at[pl.ds(s * WR, WR)])

    pltpu.sync_copy(rel_hbm, rel_v)
    base = c * HALF + s * EPT
    sets = ((col0, row0, typ0, xr0, semg0, semi0),
            (col1, row1, typ1, xr1, semg1, semi1))

    def load_idx_sync(i, st):
        b = base + i * K
        pltpu.sync_copy(col2_hbm.at[pl.ds(b, K)], st[0])
        pltpu.sync_copy(row_hbm.at[pl.ds(b, K)], st[1])
        pltpu.sync_copy(etype_hbm.at[pl.ds(b, K)], st[2])

    def load_idx_async(i, st):
        b = base + i * K
        pltpu.async_copy(col2_hbm.at[pl.ds(b, K)], st[0], st[5])
        pltpu.async_copy(row_hbm.at[pl.ds(b, K)], st[1], st[5])
        pltpu.async_copy(etype_hbm.at[pl.ds(b, K)], st[2], st[5])

    def wait_idx(i, st):
        b = base + i * K
        pltpu.make_async_copy(col2_hbm.at[pl.ds(b, K)], st[0], st[5]).wait()
        pltpu.make_async_copy(row_hbm.at[pl.ds(b, K)], st[1], st[5]).wait()
        pltpu.make_async_copy(etype_hbm.at[pl.ds(b, K)], st[2], st[5]).wait()

    def start_gather(st):
        pltpu.async_copy(xcat_hbm.at[st[0]], st[3], st[4])

    def wait_gather(st):
        pltpu.make_async_copy(xcat_hbm.at[st[0]], st[3], st[4]).wait()

    def step(i, cur, nxt):
        wait_gather(cur)

        @pl.when(i + 1 < NCHUNK)
        def _():
            @pl.when(i >= 1)
            def _():
                wait_idx(i + 1, nxt)
            start_gather(nxt)

        xr, typ = cur[3], cur[2]

        def mul_group(g, carry):
            t16 = typ[pl.ds(g * 16, 16)]
            for rr in range(16):
                t = t16[rr]
                r = g * 16 + rr
                for j in range(H // 16):
                    sl = pl.ds(j * 16, 16)
                    xr[r, sl] = xr[r, sl] * rel_v[t, sl]
            return carry

        lax.fori_loop(0, K // 16, mul_group, 0)
        pltpu.sync_copy(xr, agg_sh.at[cur[1]], add=True)

        @pl.when(i + 2 < NCHUNK)
        def _():
            load_idx_async(i + 2, cur)

    load_idx_sync(0, sets[0])
    load_idx_sync(1, sets[1])
    plsc.subcore_barrier()
    start_gather(sets[0])

    def chunk(i, carry):
        @pl.when(i % 2 == 0)
        def _():
            step(i, sets[0], sets[1])

        @pl.when(i % 2 == 1)
        def _():
            step(i, sets[1], sets[0])

        return carry

    lax.fori_loop(0, NCHUNK, chunk, 0)
    plsc.subcore_barrier()

    @pl.when(s < WT)
    def _():
        pltpu.sync_copy(agg_sh.at[pl.ds(s * WR, WR)],
                        out_hbm.at[c, pl.ds(s * WR, WR)])


_B = 1000
_NB = N // _B
_EB = E // _NB


def _scale_body(x_ref, deg_ref, col_ref, out_ref, col2_ref):
    d = deg_ref[...][:, :, 0:1]
    di = jnp.where(d > 0, lax.rsqrt(d), 0.0)
    out_ref[...] = x_ref[...][None] * di
    half_off = lax.broadcasted_iota(jnp.int32, col_ref.shape, 0) * N
    col2_ref[...] = col_ref[...] + half_off


def _scale_call(x, deg, col):
    col2d = col.reshape(NC, HALF)
    out, col2 = pl.pallas_call(
        _scale_body,
        grid=(_NB,),
        in_specs=[
            pl.BlockSpec((_B, H), lambda i: (i, 0)),
            pl.BlockSpec((NC, _B, DW), lambda i: (0, i, 0)),
            pl.BlockSpec((NC, HALF // _NB), lambda i: (0, i)),
        ],
        out_specs=[
            pl.BlockSpec((NC, _B, H), lambda i: (0, i, 0)),
            pl.BlockSpec((NC, HALF // _NB), lambda i: (0, i)),
        ],
        out_shape=[
            jax.ShapeDtypeStruct((NC, N, H), jnp.float32),
            jax.ShapeDtypeStruct((NC, HALF), jnp.int32),
        ],
    )(x, deg, col2d)
    return out, col2.reshape(E)


def _dense_body(agg_ref, deg_ref, x_ref, lrel_ref, w_in_ref, w_out_ref,
                w_loop_ref, relcat_ref, w_rel_ref,
                y_ref, stats_ref, relout_ref, acc_ref):
    i = pl.program_id(0)
    d = deg_ref[...][:, :, 0:1]
    di = jnp.where(d > 0, lax.rsqrt(d), 0.0)
    a_in = agg_ref[0] * di[0]
    a_out = agg_ref[1] * di[1]
    xl = x_ref[...] * lrel_ref[...]
    y = (jnp.dot(a_in, w_in_ref[...], preferred_element_type=jnp.float32)
         + jnp.dot(a_out, w_out_ref[...], preferred_element_type=jnp.float32)
         + jnp.dot(xl, w_loop_ref[...], preferred_element_type=jnp.float32)
         ) * (1.0 / 3.0)
    y_ref[...] = y

    @pl.when(i == 0)
    def _():
        acc_ref[...] = jnp.zeros_like(acc_ref)
        relout_ref[...] = jnp.dot(relcat_ref[...], w_rel_ref[...],
                                  preferred_element_type=jnp.float32)

    acc_ref[0, :] += jnp.sum(y, axis=0)
    acc_ref[1, :] += jnp.sum(y * y, axis=0)

    @pl.when(i == pl.num_programs(0) - 1)
    def _():
        stats_ref[...] = acc_ref[...]


def _dense_call(agg, deg, x, lrel, w_in, w_out, w_loop, relcat_p, w_rel):
    full = lambda shape: pl.BlockSpec(shape, lambda i: tuple(0 for _ in shape))
    return pl.pallas_call(
        _dense_body,
        grid=(_NB,),
        in_specs=[
            pl.BlockSpec((NC, _B, H), lambda i: (0, i, 0)),
            pl.BlockSpec((NC, _B, DW), lambda i: (0, i, 0)),
            pl.BlockSpec((_B, H), lambda i: (i, 0)),
            full((1, H)),
            full((H, H)),
            full((H, H)),
            full((H, H)),
            full(relcat_p.shape),
            full((H, H)),
        ],
        out_specs=[
            pl.BlockSpec((_B, H), lambda i: (i, 0)),
            full((2, H)),
            full(relcat_p.shape),
        ],
        out_shape=[
            jax.ShapeDtypeStruct((N, H), jnp.float32),
            jax.ShapeDtypeStruct((2, H), jnp.float32),
            jax.ShapeDtypeStruct(relcat_p.shape, jnp.float32),
        ],
        scratch_shapes=[pltpu.VMEM((2, H), jnp.float32)],
    )(agg, deg, x, lrel, w_in, w_out, w_loop, relcat_p, w_rel)


def _bn_body(y_ref, stats_ref, gamma_ref, beta_ref, out_ref):
    mean = stats_ref[0:1, :] * (1.0 / N)
    var = stats_ref[1:2, :] * (1.0 / N) - mean * mean
    inv = lax.rsqrt(var + 1e-5)
    out_ref[...] = gamma_ref[...] * ((y_ref[...] - mean) * inv) + beta_ref[...]


def _bn_call(y, stats, gamma, beta):
    full = lambda shape: pl.BlockSpec(shape, lambda i: tuple(0 for _ in shape))
    return pl.pallas_call(
        _bn_body,
        grid=(_NB,),
        in_specs=[
            pl.BlockSpec((_B, H), lambda i: (i, 0)),
            full((2, H)),
            full((1, H)),
            full((1, H)),
        ],
        out_specs=pl.BlockSpec((_B, H), lambda i: (i, 0)),
        out_shape=jax.ShapeDtypeStruct((N, H), jnp.float32),
    )(y, stats, gamma, beta)


def kernel(x, edge_index, edge_type, rel_embed, w_loop, w_in, w_out, w_rel,
           loop_rel, bn_gamma, bn_beta):
    ei = edge_index.astype(jnp.int32)
    et = edge_type.astype(jnp.int32)
    row_arr = ei[0]
    col_arr = ei[1]
    ones8 = jnp.ones((K, DW), jnp.float32)
    zeros8 = jnp.zeros((WR, DW), jnp.float32)
    deg = _deg_kernel(row_arr, ones8, zeros8)
    xcat2, col2 = _scale_call(x, deg, col_arr)
    xcat = xcat2.reshape(NC * N, H)
    zerosH = jnp.zeros((WR, H), jnp.float32)
    agg = _agg_kernel(row_arr, col2, et, xcat, rel_embed,
                      zerosH)
    relcat = jnp.concatenate([rel_embed, loop_rel], axis=0)
    relcat_p = jnp.pad(relcat, ((0, 7), (0, 0)))
    y, stats, relout = _dense_call(agg, deg, x, loop_rel, w_in, w_out,
                                   w_loop, relcat_p, w_rel)
    out = _bn_call(y, stats, bn_gamma.reshape(1, H), bn_beta.reshape(1, H))
    return out, relout[:R]


# speedup vs baseline: 1.6025x; 1.6025x over previous
import functools

import jax
import jax.numpy as jnp
from jax import lax
from jax.experimental import pallas as pl
from jax.experimental.pallas import tpu as pltpu
from jax.experimental.pallas import tpu_sc as plsc

N = 10000
E = 320000
HALF = E // 2
H = 128
R = 200
NC, NS = 2, 16
EPT = HALF // NS
K = 80
NCHUNK = EPT // K
WT = 10
WR = N // WT
SR = 40
DW = 128

_mesh = plsc.VectorSubcoreMesh(
    core_axis_name="c", subcore_axis_name="s", num_cores=NC, num_subcores=NS)


def _rsqrt16(v):
    i = lax.bitcast_convert_type(v, jnp.int32)
    i = jnp.int32(0x5F3759DF) - lax.shift_right_logical(i, 1)
    y = lax.bitcast_convert_type(i, jnp.float32)
    vh = v * 0.5
    y = y * (1.5 - vh * y * y)
    y = y * (1.5 - vh * y * y)
    y = y * (1.5 - vh * y * y)
    return jnp.where(v > 0.0, y, 0.0)


@functools.partial(
    pl.kernel,
    out_type=[
        jax.ShapeDtypeStruct((NC, N, DW), jnp.float32),
        jax.ShapeDtypeStruct((NC * N, H), jnp.float32),
    ],
    mesh=_mesh,
    scratch_types=[
        pltpu.VMEM_SHARED((N, DW), jnp.float32),
        pltpu.VMEM((EPT,), jnp.int32),
        pltpu.VMEM((K,), jnp.int32),
        pltpu.VMEM((K,), jnp.int32),
        pltpu.VMEM((K, DW), jnp.float32),
        pltpu.VMEM((SR, H), jnp.float32),
        pltpu.VMEM((SR, DW), jnp.float32),
        pltpu.SemaphoreType.DMA,
        pltpu.SemaphoreType.DMA,
    ],
)
def _deg_kernel(row_hbm, x_hbm, ones_hbm, zeros_hbm, deg_hbm, xcat_hbm,
                deg_sh, row_flat, row_v0, row_v1, ones_v, xb, db, semA, semB):
    c = lax.axis_index("c")
    s = lax.axis_index("s")

    @pl.when(s < WT)
    def _():
        pltpu.sync_copy(zeros_hbm, deg_sh.at[pl.ds(s * WR, WR)])

    pltpu.sync_copy(ones_hbm, ones_v)
    pltpu.sync_copy(row_hbm.at[pl.ds(c * HALF + s * EPT, EPT)], row_flat)
    plsc.subcore_barrier()

    def fill(dst, i):
        for j in range(K // 16):
            dst[pl.ds(j * 16, 16)] = row_flat[pl.ds(i * K + j * 16, 16)]

    def start(buf, sem):
        pltpu.async_copy(ones_v, deg_sh.at[buf], sem, add=True)

    def drain(buf, sem):
        pltpu.make_async_copy(ones_v, deg_sh.at[buf], sem).wait()

    fill(row_v0, 0)
    start(row_v0, semA)

    def chunk(i, carry):
        @pl.when(i % 2 == 0)
        def _():
            @pl.when(i + 1 < NCHUNK)
            def _():
                fill(row_v1, i + 1)
                start(row_v1, semB)
            drain(row_v0, semA)

        @pl.when(i % 2 == 1)
        def _():
            @pl.when(i + 1 < NCHUNK)
            def _():
                fill(row_v0, i + 1)
                start(row_v0, semA)
            drain(row_v1, semB)

        return carry

    lax.fori_loop(0, NCHUNK, chunk, 0)
    plsc.subcore_barrier()

    @pl.when(s < WT)
    def _():
        pltpu.sync_copy(deg_sh.at[pl.ds(s * WR, WR)],
                        deg_hbm.at[c, pl.ds(s * WR, WR)])

        def scale_chunk(i, carry):
            r0 = s * WR + i * SR
            pltpu.sync_copy(x_hbm.at[pl.ds(r0, SR)], xb)
            pltpu.sync_copy(deg_sh.at[pl.ds(r0, SR)], db)

            def srow(r, carry2):
                d = _rsqrt16(db[r, pl.ds(0, 16)])
                for j in range(H // 16):
                    sl = pl.ds(j * 16, 16)
                    xb[r, sl] = xb[r, sl] * d
                return carry2

            lax.fori_loop(0, SR, srow, 0)
            pltpu.sync_copy(xb, xcat_hbm.at[pl.ds(c * N + r0, SR)])
            return carry

        lax.fori_loop(0, WR // SR, scale_chunk, 0)


@functools.partial(
    pl.kernel,
    out_type=jax.ShapeDtypeStruct((NC, N, H), jnp.float32),
    mesh=_mesh,
    scratch_types=[
        pltpu.VMEM_SHARED((N, H), jnp.float32),
        pltpu.VMEM((K,), jnp.int32),
        pltpu.VMEM((K,), jnp.int32),
        pltpu.VMEM((K,), jnp.int32),
        pltpu.VMEM((K,), jnp.int32),
        pltpu.VMEM((K,), jnp.int32),
        pltpu.VMEM((K,), jnp.int32),
        pltpu.VMEM((K, H), jnp.float32),
        pltpu.VMEM((K, H), jnp.float32),
        pltpu.VMEM((K, H), jnp.float32),
        pltpu.VMEM((K, H), jnp.float32),
        pltpu.SemaphoreType.DMA,
        pltpu.SemaphoreType.DMA,
        pltpu.SemaphoreType.DMA,
        pltpu.SemaphoreType.DMA,
    ],
)
def _agg_kernel(row_hbm, col_hbm, etype_hbm, xcat_hbm, rel_hbm, zeros_hbm,
                out_hbm, agg_sh, col0, row0, typ0, col1, row1, typ1,
                xr0, rr0, xr1, rr1, semg0, semg1, semi0, semi1):
    c = lax.axis_index("c")
    s = lax.axis_index("s")

    @pl.when(s < WT)
    def _():
        pltpu.sync_copy(zeros_hbm, agg_sh.at[pl.ds(s * WR, WR)])

    base = c * HALF + s * EPT
    off = c * N
    sets = ((col0, row0, typ0, xr0, rr0, semg0, semi0),
            (col1, row1, typ1, xr1, rr1, semg1, semi1))

    def add_off(st):
        for j in range(K // 16):
            sl = pl.ds(j * 16, 16)
            st[0][sl] = st[0][sl] + off

    def load_idx_sync(i, st):
        b = base + i * K
        pltpu.sync_copy(col_hbm.at[pl.ds(b, K)], st[0])
        pltpu.sync_copy(row_hbm.at[pl.ds(b, K)], st[1])
        pltpu.sync_copy(etype_hbm.at[pl.ds(b, K)], st[2])
        add_off(st)

    def load_idx_async(i, st):
        b = base + i * K
        pltpu.async_copy(col_hbm.at[pl.ds(b, K)], st[0], st[6])
        pltpu.async_copy(row_hbm.at[pl.ds(b, K)], st[1], st[6])
        pltpu.async_copy(etype_hbm.at[pl.ds(b, K)], st[2], st[6])

    def wait_idx(i, st):
        b = base + i * K
        pltpu.make_async_copy(col_hbm.at[pl.ds(b, K)], st[0], st[6]).wait()
        pltpu.make_async_copy(row_hbm.at[pl.ds(b, K)], st[1], st[6]).wait()
        pltpu.make_async_copy(etype_hbm.at[pl.ds(b, K)], st[2], st[6]).wait()
        add_off(st)

    def start_gather(st):
        pltpu.async_copy(xcat_hbm.at[st[0]], st[3], st[5])
        pltpu.async_copy(rel_hbm.at[st[2]], st[4], st[5])

    def wait_gather(st):
        pltpu.make_async_copy(xcat_hbm.at[st[0]], st[3], st[5]).wait()
        pltpu.make_async_copy(rel_hbm.at[st[2]], st[4], st[5]).wait()

    def step(i, cur, nxt):
        wait_gather(cur)

        @pl.when(i + 1 < NCHUNK)
        def _():
            @pl.when(i >= 1)
            def _():
                wait_idx(i + 1, nxt)
            start_gather(nxt)

        xr, rr = cur[3], cur[4]

        def mul_row(r, carry):
            for j in range(H // 16):
                sl = pl.ds(j * 16, 16)
                xr[r, sl] = xr[r, sl] * rr[r, sl]
            return carry

        lax.fori_loop(0, K, mul_row, 0)
        pltpu.sync_copy(xr, agg_sh.at[cur[1]], add=True)

        @pl.when(i + 2 < NCHUNK)
        def _():
            load_idx_async(i + 2, cur)

    load_idx_sync(0, sets[0])
    load_idx_sync(1, sets[1])
    plsc.subcore_barrier()
    start_gather(sets[0])

    def chunk(i, carry):
        @pl.when(i % 2 == 0)
        def _():
            step(i, sets[0], sets[1])

        @pl.when(i % 2 == 1)
        def _():
            step(i, sets[1], sets[0])

        return carry

    lax.fori_loop(0, NCHUNK, chunk, 0)
    plsc.subcore_barrier()

    @pl.when(s < WT)
    def _():
        pltpu.sync_copy(agg_sh.at[pl.ds(s * WR, WR)],
                        out_hbm.at[c, pl.ds(s * WR, WR)])


_B = 1000
_NB = N // _B


def _dense_body(agg_ref, deg_ref, x_ref, lrel_ref, w_in_ref, w_out_ref,
                w_loop_ref, relcat_ref, w_rel_ref, gamma_ref, beta_ref,
                out_ref, relout_ref, y_scr, acc_ref):
    p = pl.program_id(0)
    i = pl.program_id(1)

    @pl.when(p == 0)
    def _():
        d = deg_ref[...][:, :, 0:1]
        di = jnp.where(d > 0, lax.rsqrt(d), 0.0)
        a_in = agg_ref[0] * di[0]
        a_out = agg_ref[1] * di[1]
        xl = x_ref[...] * lrel_ref[...]
        y = (jnp.dot(a_in, w_in_ref[...], preferred_element_type=jnp.float32)
             + jnp.dot(a_out, w_out_ref[...],
                       preferred_element_type=jnp.float32)
             + jnp.dot(xl, w_loop_ref[...],
                       preferred_element_type=jnp.float32)) * (1.0 / 3.0)
        y_scr[pl.ds(i * _B, _B), :] = y

        @pl.when(i == 0)
        def _():
            acc_ref[...] = jnp.zeros_like(acc_ref)
            relout_ref[...] = jnp.dot(relcat_ref[...], w_rel_ref[...],
                                      preferred_element_type=jnp.float32)

        acc_ref[0, :] += jnp.sum(y, axis=0)
        acc_ref[1, :] += jnp.sum(y * y, axis=0)

    @pl.when(p == 1)
    def _():
        mean = acc_ref[0:1, :] * (1.0 / N)
        var = acc_ref[1:2, :] * (1.0 / N) - mean * mean
        inv = lax.rsqrt(var + 1e-5)
        y = y_scr[pl.ds(i * _B, _B), :]
        out_ref[...] = gamma_ref[...] * ((y - mean) * inv) + beta_ref[...]


def _dense_call(agg, deg, x, lrel, w_in, w_out, w_loop, relcat_p, w_rel,
                gamma, beta):
    full = lambda shape: pl.BlockSpec(
        shape, lambda p, i: tuple(0 for _ in shape))
    return pl.pallas_call(
        _dense_body,
        grid=(2, _NB),
        in_specs=[
            pl.BlockSpec((NC, _B, H), lambda p, i: (0, i * (1 - p), 0)),
            pl.BlockSpec((NC, _B, DW), lambda p, i: (0, i * (1 - p), 0)),
            pl.BlockSpec((_B, H), lambda p, i: (i * (1 - p), 0)),
            full((1, H)),
            full((H, H)),
            full((H, H)),
            full((H, H)),
            full(relcat_p.shape),
            full((H, H)),
            full((1, H)),
            full((1, H)),
        ],
        out_specs=[
            pl.BlockSpec((_B, H), lambda p, i: (i, 0)),
            full(relcat_p.shape),
        ],
        out_shape=[
            jax.ShapeDtypeStruct((N, H), jnp.float32),
            jax.ShapeDtypeStruct(relcat_p.shape, jnp.float32),
        ],
        scratch_shapes=[
            pltpu.VMEM((N, H), jnp.float32),
            pltpu.VMEM((2, H), jnp.float32),
        ],
    )(agg, deg, x, lrel, w_in, w_out, w_loop, relcat_p, w_rel, gamma, beta)


def kernel(x, edge_index, edge_type, rel_embed, w_loop, w_in, w_out, w_rel,
           loop_rel, bn_gamma, bn_beta):
    ei = edge_index.astype(jnp.int32)
    et = edge_type.astype(jnp.int32)
    row_arr = ei[0]
    col_arr = ei[1]
    ones8 = jnp.ones((K, DW), jnp.float32)
    zeros8 = jnp.zeros((WR, DW), jnp.float32)
    deg, xcat = _deg_kernel(row_arr, x, ones8, zeros8)
    zerosH = jnp.zeros((WR, H), jnp.float32)
    agg = _agg_kernel(row_arr, col_arr, et, xcat, rel_embed,
                      zerosH)
    relcat = jnp.concatenate([rel_embed, loop_rel], axis=0)
    relcat_p = jnp.pad(relcat, ((0, 7), (0, 0)))
    out, relout = _dense_call(agg, deg, x, loop_rel, w_in, w_out, w_loop,
                              relcat_p, w_rel, bn_gamma.reshape(1, H),
                              bn_beta.reshape(1, H))
    return out, relout[:R]


# speedup vs baseline: 1.6930x; 1.0565x over previous
import functools

import jax
import jax.numpy as jnp
from jax import lax
from jax.experimental import pallas as pl
from jax.experimental.pallas import tpu as pltpu
from jax.experimental.pallas import tpu_sc as plsc

N = 10000
E = 320000
HALF = E // 2
H = 128
R = 200
NC, NS = 2, 16
EPT = HALF // NS
K = 80
NCHUNK = EPT // K
WT = 10
WR = N // WT
SR = 80
NJ = N // SR
RP = 208
DW = 128

_mesh = plsc.VectorSubcoreMesh(
    core_axis_name="c", subcore_axis_name="s", num_cores=NC, num_subcores=NS)


def _rsqrt16(v):
    i = lax.bitcast_convert_type(v, jnp.int32)
    i = jnp.int32(0x5F3759DF) - lax.shift_right_logical(i, 1)
    y = lax.bitcast_convert_type(i, jnp.float32)
    vh = v * 0.5
    y = y * (1.5 - vh * y * y)
    y = y * (1.5 - vh * y * y)
    y = y * (1.5 - vh * y * y)
    return jnp.where(v > 0.0, y, 0.0)


@functools.partial(
    pl.kernel,
    out_type=[
        jax.ShapeDtypeStruct((NC, N, DW), jnp.float32),
        jax.ShapeDtypeStruct((NC * N, H), jnp.float32),
    ],
    mesh=_mesh,
    scratch_types=[
        pltpu.VMEM_SHARED((N, DW), jnp.float32),
        pltpu.VMEM((EPT,), jnp.int32),
        pltpu.VMEM((K,), jnp.int32),
        pltpu.VMEM((K,), jnp.int32),
        pltpu.VMEM((K, DW), jnp.float32),
        pltpu.VMEM((SR, H), jnp.float32),
        pltpu.VMEM((SR, DW), jnp.float32),
        pltpu.SemaphoreType.DMA,
        pltpu.SemaphoreType.DMA,
    ],
)
def _deg_kernel(row_hbm, x_hbm, ones_hbm, zeros_hbm,
                deg_hbm, xcat_hbm,
                deg_sh, row_flat, row_v0, row_v1, ones_v, xb, db,
                semA, semB):
    c = lax.axis_index("c")
    s = lax.axis_index("s")

    @pl.when(s < WT)
    def _():
        pltpu.sync_copy(zeros_hbm, deg_sh.at[pl.ds(s * WR, WR)])

    pltpu.sync_copy(ones_hbm, ones_v)
    pltpu.sync_copy(row_hbm.at[pl.ds(c * HALF + s * EPT, EPT)], row_flat)
    plsc.subcore_barrier()

    def fill(dst, i):
        for j in range(K // 16):
            dst[pl.ds(j * 16, 16)] = row_flat[pl.ds(i * K + j * 16, 16)]

    def start(buf, sem):
        pltpu.async_copy(ones_v, deg_sh.at[buf], sem, add=True)

    def drain(buf, sem):
        pltpu.make_async_copy(ones_v, deg_sh.at[buf], sem).wait()

    fill(row_v0, 0)
    start(row_v0, semA)

    def chunk(i, carry):
        @pl.when(i % 2 == 0)
        def _():
            @pl.when(i + 1 < NCHUNK)
            def _():
                fill(row_v1, i + 1)
                start(row_v1, semB)
            drain(row_v0, semA)

        @pl.when(i % 2 == 1)
        def _():
            @pl.when(i + 1 < NCHUNK)
            def _():
                fill(row_v0, i + 1)
                start(row_v0, semA)
            drain(row_v1, semB)

        return carry

    lax.fori_loop(0, NCHUNK, chunk, 0)
    plsc.subcore_barrier()

    def scale_chunk(it, carry):
        j = it * NS + s

        @pl.when(j < NJ)
        def _():
            r0 = j * SR
            pltpu.sync_copy(x_hbm.at[pl.ds(r0, SR)], xb)
            pltpu.sync_copy(deg_sh.at[pl.ds(r0, SR)], db)

            def srow(r, carry2):
                d = _rsqrt16(db[r, pl.ds(0, 16)])
                for jj in range(H // 16):
                    sl = pl.ds(jj * 16, 16)
                    xb[r, sl] = xb[r, sl] * d
                return carry2

            lax.fori_loop(0, SR, srow, 0)
            pltpu.sync_copy(xb, xcat_hbm.at[pl.ds(c * N + r0, SR)])
            pltpu.sync_copy(db, deg_hbm.at[c, pl.ds(r0, SR)])

        return carry

    lax.fori_loop(0, (NJ + NS - 1) // NS, scale_chunk, 0)


@functools.partial(
    pl.kernel,
    out_type=jax.ShapeDtypeStruct((NC, N, H), jnp.float32),
    mesh=_mesh,
    scratch_types=[
        pltpu.VMEM_SHARED((N, H), jnp.float32),
        pltpu.VMEM((K,), jnp.int32),
        pltpu.VMEM((K,), jnp.int32),
        pltpu.VMEM((K,), jnp.int32),
        pltpu.VMEM((K,), jnp.int32),
        pltpu.VMEM((K,), jnp.int32),
        pltpu.VMEM((K,), jnp.int32),
        pltpu.VMEM((K, H), jnp.float32),
        pltpu.VMEM((K, H), jnp.float32),
        pltpu.VMEM((K, H), jnp.float32),
        pltpu.VMEM((K, H), jnp.float32),
        pltpu.SemaphoreType.DMA,
        pltpu.SemaphoreType.DMA,
        pltpu.SemaphoreType.DMA,
        pltpu.SemaphoreType.DMA,
    ],
)
def _agg_kernel(row_hbm, col_hbm, etype_hbm, xcat_hbm, rel_hbm, zeros_hbm,
                out_hbm, agg_sh, col0, row0, typ0, col1, row1, typ1,
                xr0, rr0, xr1, rr1, semg0, semg1, semi0, semi1):
    c = lax.axis_index("c")
    s = lax.axis_index("s")

    @pl.when(s < WT)
    def _():
        pltpu.sync_copy(zeros_hbm, agg_sh.at[pl.ds(s * WR, WR)])

    base = c * HALF + s * EPT
    off = c * N
    sets = ((col0, row0, typ0, xr0, rr0, semg0, semi0),
            (col1, row1, typ1, xr1, rr1, semg1, semi1))

    def add_off(st):
        for j in range(K // 16):
            sl = pl.ds(j * 16, 16)
            st[0][sl] = st[0][sl] + off

    def load_idx_sync(i, st):
        b = base + i * K
        pltpu.sync_copy(col_hbm.at[pl.ds(b, K)], st[0])
        pltpu.sync_copy(row_hbm.at[pl.ds(b, K)], st[1])
        pltpu.sync_copy(etype_hbm.at[pl.ds(b, K)], st[2])
        add_off(st)

    def load_idx_async(i, st):
        b = base + i * K
        pltpu.async_copy(col_hbm.at[pl.ds(b, K)], st[0], st[6])
        pltpu.async_copy(row_hbm.at[pl.ds(b, K)], st[1], st[6])
        pltpu.async_copy(etype_hbm.at[pl.ds(b, K)], st[2], st[6])

    def wait_idx(i, st):
        b = base + i * K
        pltpu.make_async_copy(col_hbm.at[pl.ds(b, K)], st[0], st[6]).wait()
        pltpu.make_async_copy(row_hbm.at[pl.ds(b, K)], st[1], st[6]).wait()
        pltpu.make_async_copy(etype_hbm.at[pl.ds(b, K)], st[2], st[6]).wait()
        add_off(st)

    def start_gather(st):
        pltpu.async_copy(xcat_hbm.at[st[0]], st[3], st[5])
        pltpu.async_copy(rel_hbm.at[st[2]], st[4], st[5])

    def wait_gather(st):
        pltpu.make_async_copy(xcat_hbm.at[st[0]], st[3], st[5]).wait()
        pltpu.make_async_copy(rel_hbm.at[st[2]], st[4], st[5]).wait()

    def step(i, cur, nxt):
        wait_gather(cur)

        @pl.when(i + 1 < NCHUNK)
        def _():
            @pl.when(i >= 1)
            def _():
                wait_idx(i + 1, nxt)
            start_gather(nxt)

        xr, rr = cur[3], cur[4]

        def mul_row(r, carry):
            for j in range(H // 16):
                sl = pl.ds(j * 16, 16)
                xr[r, sl] = xr[r, sl] * rr[r, sl]
            return carry

        lax.fori_loop(0, K, mul_row, 0)
        pltpu.sync_copy(xr, agg_sh.at[cur[1]], add=True)

        @pl.when(i + 2 < NCHUNK)
        def _():
            load_idx_async(i + 2, cur)

    load_idx_sync(0, sets[0])
    load_idx_sync(1, sets[1])
    plsc.subcore_barrier()
    start_gather(sets[0])

    def chunk(i, carry):
        @pl.when(i % 2 == 0)
        def _():
            step(i, sets[0], sets[1])

        @pl.when(i % 2 == 1)
        def _():
            step(i, sets[1], sets[0])

        return carry

    lax.fori_loop(0, NCHUNK, chunk, 0)
    plsc.subcore_barrier()

    @pl.when(s < WT)
    def _():
        pltpu.sync_copy(agg_sh.at[pl.ds(s * WR, WR)],
                        out_hbm.at[c, pl.ds(s * WR, WR)])


_B = 1000
_NB = N // _B


def _dense_body(agg_ref, deg_ref, x_ref, lrel_ref, w_in_ref, w_out_ref,
                w_loop_ref, relcat_ref, w_rel_ref, gamma_ref, beta_ref,
                out_ref, relout_ref, y_scr, acc_ref):
    p = pl.program_id(0)
    i = pl.program_id(1)

    @pl.when(p == 0)
    def _():
        d = deg_ref[...][:, :, 0:1]
        di = jnp.where(d > 0, lax.rsqrt(d), 0.0)
        a_in = agg_ref[0] * di[0]
        a_out = agg_ref[1] * di[1]
        xl = x_ref[...] * lrel_ref[...]
        y = (jnp.dot(a_in, w_in_ref[...], preferred_element_type=jnp.float32)
             + jnp.dot(a_out, w_out_ref[...],
                       preferred_element_type=jnp.float32)
             + jnp.dot(xl, w_loop_ref[...],
                       preferred_element_type=jnp.float32)) * (1.0 / 3.0)
        y_scr[pl.ds(i * _B, _B), :] = y

        @pl.when(i == 0)
        def _():
            acc_ref[...] = jnp.zeros_like(acc_ref)
            relout_ref[...] = jnp.dot(relcat_ref[...], w_rel_ref[...],
                                      preferred_element_type=jnp.float32)

        acc_ref[0, :] += jnp.sum(y, axis=0)
        acc_ref[1, :] += jnp.sum(y * y, axis=0)

    @pl.when(p == 1)
    def _():
        mean = acc_ref[0:1, :] * (1.0 / N)
        var = acc_ref[1:2, :] * (1.0 / N) - mean * mean
        inv = lax.rsqrt(var + 1e-5)
        y = y_scr[pl.ds(i * _B, _B), :]
        out_ref[...] = gamma_ref[...] * ((y - mean) * inv) + beta_ref[...]


def _dense_call(agg, deg, x, lrel, w_in, w_out, w_loop, relcat_p, w_rel,
                gamma, beta):
    full = lambda shape: pl.BlockSpec(
        shape, lambda p, i: tuple(0 for _ in shape))
    return pl.pallas_call(
        _dense_body,
        grid=(2, _NB),
        in_specs=[
            pl.BlockSpec((NC, _B, H), lambda p, i: (0, i * (1 - p), 0)),
            pl.BlockSpec((NC, _B, DW), lambda p, i: (0, i * (1 - p), 0)),
            pl.BlockSpec((_B, H), lambda p, i: (i * (1 - p), 0)),
            full((1, H)),
            full((H, H)),
            full((H, H)),
            full((H, H)),
            full(relcat_p.shape),
            full((H, H)),
            full((1, H)),
            full((1, H)),
        ],
        out_specs=[
            pl.BlockSpec((_B, H), lambda p, i: (i, 0)),
            full(relcat_p.shape),
        ],
        out_shape=[
            jax.ShapeDtypeStruct((N, H), jnp.float32),
            jax.ShapeDtypeStruct(relcat_p.shape, jnp.float32),
        ],
        scratch_shapes=[
            pltpu.VMEM((N, H), jnp.float32),
            pltpu.VMEM((2, H), jnp.float32),
        ],
    )(agg, deg, x, lrel, w_in, w_out, w_loop, relcat_p, w_rel, gamma, beta)


def kernel(x, edge_index, edge_type, rel_embed, w_loop, w_in, w_out, w_rel,
           loop_rel, bn_gamma, bn_beta):
    ei = edge_index.astype(jnp.int32)
    et = edge_type.astype(jnp.int32)
    row_arr = ei[0]
    col_arr = ei[1]
    ones8 = jnp.ones((K, DW), jnp.float32)
    zeros8 = jnp.zeros((WR, DW), jnp.float32)
    deg, xcat = _deg_kernel(row_arr, x, ones8, zeros8)
    zerosH = jnp.zeros((WR, H), jnp.float32)
    agg = _agg_kernel(row_arr, col_arr, et, xcat, rel_embed,
                      zerosH)
    relcat = jnp.concatenate([rel_embed, loop_rel], axis=0)
    relcat_p = jnp.pad(relcat, ((0, 7), (0, 0)))
    out, relout = _dense_call(agg, deg, x, loop_rel, w_in, w_out, w_loop,
                              relcat_p, w_rel, bn_gamma.reshape(1, H),
                              bn_beta.reshape(1, H))
    return out, relout[:R]
